# baseline (speedup 1.0000x reference)
import functools

import jax
import jax.numpy as jnp
from jax import lax
from jax.experimental import pallas as pl
from jax.experimental.pallas import tpu as pltpu
from jax.experimental.pallas import tpu_sc as plsc

_NC = 2
_NS = 16


def _mm_body(at_ref, bt_ref, o_ref):
    o_ref[...] = lax.dot_general(
        at_ref[...], bt_ref[...],
        dimension_numbers=(((0,), (0,)), ((), ())),
        preferred_element_type=jnp.float32,
    )


def _project_table(At, Bt, block_cols):
    k, vocab = At.shape
    d = Bt.shape[1]
    grid = (vocab + block_cols - 1) // block_cols
    return pl.pallas_call(
        _mm_body,
        grid=(grid,),
        in_specs=[
            pl.BlockSpec((k, block_cols), lambda i: (0, i)),
            pl.BlockSpec((k, d), lambda i: (0, 0)),
        ],
        out_specs=pl.BlockSpec((block_cols, d), lambda i: (i, 0)),
        out_shape=jax.ShapeDtypeStruct((vocab, d), jnp.float32),
    )(At, Bt)


def _make_gather(n_idx, d, per_w, chunk, nbuf=3):
    n_chunks = per_w // chunk
    mesh = plsc.VectorSubcoreMesh(core_axis_name="c", subcore_axis_name="s")

    @functools.partial(
        pl.kernel,
        out_type=jax.ShapeDtypeStruct((n_idx, d), jnp.float32),
        mesh=mesh,
        scratch_types=[
            pltpu.VMEM((per_w,), jnp.int32),
        ] + [pltpu.VMEM((chunk, d), jnp.float32) for _ in range(nbuf)]
          + [pltpu.SemaphoreType.DMA for _ in range(2 * nbuf)],
    )
    def gather(table_hbm, idx_hbm, out_hbm, idx_v, *bufs_sems):
        bufs = bufs_sems[:nbuf]
        gsems = bufs_sems[nbuf:2 * nbuf]
        wsems = bufs_sems[2 * nbuf:]
        wid = lax.axis_index("s") * _NC + lax.axis_index("c")
        base = wid * per_w
        pltpu.sync_copy(idx_hbm.at[pl.ds(base, per_w)], idx_v)

        def fire_gather(i, s):
            pltpu.async_copy(
                table_hbm.at[idx_v.at[pl.ds(i * chunk, chunk)]],
                bufs[s], gsems[s])

        def wait_gather(i, s):
            pltpu.make_async_copy(
                table_hbm.at[idx_v.at[pl.ds(i * chunk, chunk)]],
                bufs[s], gsems[s]).wait()

        def fire_write(i, s):
            pltpu.async_copy(
                bufs[s], out_hbm.at[pl.ds(base + i * chunk, chunk)], wsems[s])

        def wait_write(i, s):
            pltpu.make_async_copy(
                bufs[s], out_hbm.at[pl.ds(base + i * chunk, chunk)],
                wsems[s]).wait()

        ahead = nbuf - 1
        for j in range(ahead):
            fire_gather(j, j)

        def step(i, _):
            for s in range(nbuf):
                @pl.when(lax.rem(i, nbuf) == s)
                def _():
                    s2 = (s + ahead) % nbuf
                    @pl.when(i + ahead < n_chunks)
                    def _():
                        @pl.when(i >= 1)
                        def _():
                            wait_write(i - 1, s2)
                        fire_gather(i + ahead, s2)
                    wait_gather(i, s)
                    fire_write(i, s)
            return 0

        lax.fori_loop(0, n_chunks, step, 0)

        for j in range(max(0, n_chunks - nbuf), n_chunks):
            wait_write(j, j % nbuf)

    return gather


def kernel(token_ids, A, B):
    batch, hist = token_ids.shape
    vocab, k = A.shape
    d = B.shape[0]

    table = _project_table(A.T, B.T, block_cols=12800)

    n_idx = batch * hist
    idx_hm = token_ids.T.reshape(n_idx).astype(jnp.int32)

    per_w = n_idx // (_NC * _NS)
    gather = _make_gather(n_idx, d, per_w, chunk=200, nbuf=4)
    low = gather(table, idx_hm)

    return low.reshape(hist, batch, d).transpose(1, 0, 2)


# speedup vs baseline: 1.0029x; 1.0029x over previous
import functools

import jax
import jax.numpy as jnp
from jax import lax
from jax.experimental import pallas as pl
from jax.experimental.pallas import tpu as pltpu
from jax.experimental.pallas import tpu_sc as plsc

_NC = 2
_NS = 16


def _mm_body(at_ref, bt_ref, o_ref):
    o_ref[...] = lax.dot_general(
        at_ref[...], bt_ref[...],
        dimension_numbers=(((0,), (0,)), ((), ())),
        preferred_element_type=jnp.float32,
    )


def _project_table(At, Bt, block_cols):
    k, vocab = At.shape
    d = Bt.shape[1]
    grid = (vocab + block_cols - 1) // block_cols
    return pl.pallas_call(
        _mm_body,
        grid=(grid,),
        in_specs=[
            pl.BlockSpec((k, block_cols), lambda i: (0, i)),
            pl.BlockSpec((k, d), lambda i: (0, 0)),
        ],
        out_specs=pl.BlockSpec((block_cols, d), lambda i: (i, 0)),
        out_shape=jax.ShapeDtypeStruct((vocab, d), jnp.float32),
    )(At, Bt)


def _make_gather(n_idx, d, per_w, chunk, nbuf=3):
    n_chunks = per_w // chunk
    mesh = plsc.VectorSubcoreMesh(core_axis_name="c", subcore_axis_name="s")

    @functools.partial(
        pl.kernel,
        out_type=jax.ShapeDtypeStruct((n_idx, d), jnp.float32),
        mesh=mesh,
        scratch_types=[
            pltpu.VMEM((per_w,), jnp.int32),
        ] + [pltpu.VMEM((2, chunk // 2, d), jnp.float32) for _ in range(nbuf)]
          + [pltpu.SemaphoreType.DMA for _ in range(2 * nbuf)],
    )
    def gather(table_hbm, idx_hbm, out_hbm, idx_v, *bufs_sems):
        bufs = bufs_sems[:nbuf]
        gsems = bufs_sems[nbuf:2 * nbuf]
        wsems = bufs_sems[2 * nbuf:]
        wid = lax.axis_index("s") * _NC + lax.axis_index("c")
        base = wid * per_w
        half = chunk // 2
        pltpu.sync_copy(idx_hbm.at[pl.ds(base, per_w)], idx_v)

        def fire_gather(i, s):
            pltpu.async_copy(
                table_hbm.at[idx_v.at[pl.ds(i * chunk, half)]],
                bufs[s].at[0], gsems[s])
            pltpu.async_copy(
                table_hbm.at[idx_v.at[pl.ds(i * chunk + half, half)]],
                bufs[s].at[1], gsems[s])

        def wait_gather(i, s):
            pltpu.make_async_copy(
                table_hbm.at[idx_v.at[pl.ds(i * chunk, half)]],
                bufs[s].at[0], gsems[s]).wait()
            pltpu.make_async_copy(
                table_hbm.at[idx_v.at[pl.ds(i * chunk + half, half)]],
                bufs[s].at[1], gsems[s]).wait()

        def fire_write(i, s):
            pltpu.async_copy(
                bufs[s].at[0], out_hbm.at[pl.ds(base + i * chunk, half)],
                wsems[s])
            pltpu.async_copy(
                bufs[s].at[1],
                out_hbm.at[pl.ds(base + i * chunk + half, half)], wsems[s])

        def wait_write(i, s):
            pltpu.make_async_copy(
                bufs[s].at[0], out_hbm.at[pl.ds(base + i * chunk, half)],
                wsems[s]).wait()
            pltpu.make_async_copy(
                bufs[s].at[1],
                out_hbm.at[pl.ds(base + i * chunk + half, half)],
                wsems[s]).wait()

        ahead = nbuf - 1
        for j in range(ahead):
            fire_gather(j, j)

        def step(i, _):
            for s in range(nbuf):
                @pl.when(lax.rem(i, nbuf) == s)
                def _():
                    s2 = (s + ahead) % nbuf
                    @pl.when(i + ahead < n_chunks)
                    def _():
                        @pl.when(i >= 1)
                        def _():
                            wait_write(i - 1, s2)
                        fire_gather(i + ahead, s2)
                    wait_gather(i, s)
                    fire_write(i, s)
            return 0

        lax.fori_loop(0, n_chunks, step, 0)

        for j in range(max(0, n_chunks - nbuf), n_chunks):
            wait_write(j, j % nbuf)

    return gather


def kernel(token_ids, A, B):
    batch, hist = token_ids.shape
    vocab, k = A.shape
    d = B.shape[0]

    table = _project_table(A.T, B.T, block_cols=12800)

    n_idx = batch * hist
    idx_hm = token_ids.T.reshape(n_idx).astype(jnp.int32)

    per_w = n_idx // (_NC * _NS)
    gather = _make_gather(n_idx, d, per_w, chunk=320, nbuf=3)
    low = gather(table, idx_hm)

    return low.reshape(hist, batch, d).transpose(1, 0, 2)


# speedup vs baseline: 1.0086x; 1.0057x over previous
import functools

import jax
import jax.numpy as jnp
from jax import lax
from jax.experimental import pallas as pl
from jax.experimental.pallas import tpu as pltpu
from jax.experimental.pallas import tpu_sc as plsc

_NC = 2
_NS = 16


def _mm_body(at_ref, bt_ref, o_ref):
    o_ref[...] = lax.dot_general(
        at_ref[...], bt_ref[...],
        dimension_numbers=(((0,), (0,)), ((), ())),
        preferred_element_type=jnp.float32,
    )


def _project_table(At, Bt, block_cols):
    k, vocab = At.shape
    d = Bt.shape[1]
    grid = (vocab + block_cols - 1) // block_cols
    return pl.pallas_call(
        _mm_body,
        grid=(grid,),
        in_specs=[
            pl.BlockSpec((k, block_cols), lambda i: (0, i)),
            pl.BlockSpec((k, d), lambda i: (0, 0)),
        ],
        out_specs=pl.BlockSpec((block_cols, d), lambda i: (i, 0)),
        out_shape=jax.ShapeDtypeStruct((vocab, d), jnp.float32),
    )(At, Bt)


def _make_gather(n_idx, d, per_w, chunk, nbuf=3):
    n_chunks = per_w // chunk
    mesh = plsc.VectorSubcoreMesh(core_axis_name="c", subcore_axis_name="s")

    @functools.partial(
        pl.kernel,
        out_type=jax.ShapeDtypeStruct((n_idx, d), jnp.float32),
        mesh=mesh,
        scratch_types=[
            pltpu.VMEM((per_w,), jnp.int32),
        ] + [pltpu.VMEM((2, chunk // 2, d), jnp.float32) for _ in range(nbuf)]
          + [pltpu.SemaphoreType.DMA for _ in range(2 * nbuf)],
    )
    def gather(table_hbm, idx_hbm, out_hbm, idx_v, *bufs_sems):
        bufs = bufs_sems[:nbuf]
        gsems = bufs_sems[nbuf:2 * nbuf]
        wsems = bufs_sems[2 * nbuf:]
        wid = lax.axis_index("s") * _NC + lax.axis_index("c")
        base = wid * per_w
        half = chunk // 2
        pltpu.sync_copy(idx_hbm.at[pl.ds(base, per_w)], idx_v)

        def fire_gather(i, s):
            pltpu.async_copy(
                table_hbm.at[idx_v.at[pl.ds(i * chunk, half)]],
                bufs[s].at[0], gsems[s])
            pltpu.async_copy(
                table_hbm.at[idx_v.at[pl.ds(i * chunk + half, half)]],
                bufs[s].at[1], gsems[s])

        def wait_gather(i, s):
            pltpu.make_async_copy(
                table_hbm.at[idx_v.at[pl.ds(i * chunk, half)]],
                bufs[s].at[0], gsems[s]).wait()
            pltpu.make_async_copy(
                table_hbm.at[idx_v.at[pl.ds(i * chunk + half, half)]],
                bufs[s].at[1], gsems[s]).wait()

        def fire_write(i, s):
            pltpu.async_copy(
                bufs[s].at[0], out_hbm.at[pl.ds(base + i * chunk, half)],
                wsems[s])
            pltpu.async_copy(
                bufs[s].at[1],
                out_hbm.at[pl.ds(base + i * chunk + half, half)], wsems[s])

        def wait_write(i, s):
            pltpu.make_async_copy(
                bufs[s].at[0], out_hbm.at[pl.ds(base + i * chunk, half)],
                wsems[s]).wait()
            pltpu.make_async_copy(
                bufs[s].at[1],
                out_hbm.at[pl.ds(base + i * chunk + half, half)],
                wsems[s]).wait()

        ahead = nbuf - 1
        for j in range(ahead):
            fire_gather(j, j)

        def step(i, _):
            for s in range(nbuf):
                @pl.when(lax.rem(i, nbuf) == s)
                def _():
                    s2 = (s + ahead) % nbuf
                    @pl.when(i + ahead < n_chunks)
                    def _():
                        @pl.when(i >= 1)
                        def _():
                            wait_write(i - 1, s2)
                        fire_gather(i + ahead, s2)
                    wait_gather(i, s)
                    fire_write(i, s)
            return 0

        lax.fori_loop(0, n_chunks, step, 0)

        for j in range(max(0, n_chunks - nbuf), n_chunks):
            wait_write(j, j % nbuf)

    return gather


def kernel(token_ids, A, B):
    batch, hist = token_ids.shape
    vocab, k = A.shape
    d = B.shape[0]

    table = _project_table(A.T, B.T, block_cols=25600)

    n_idx = batch * hist
    idx_hm = token_ids.T.reshape(n_idx).astype(jnp.int32)

    per_w = n_idx // (_NC * _NS)
    gather = _make_gather(n_idx, d, per_w, chunk=320, nbuf=3)
    low = gather(table, idx_hm)

    return low.reshape(hist, batch, d).transpose(1, 0, 2)
